# baseline (speedup 1.0000x reference)
import jax
import jax.numpy as jnp
from jax.experimental import pallas as pl

_ROWS = 1536


def _pool_half(t, p):
    w = t.shape[1] // 2
    a = t[:, :w]
    b = t[:, w:]
    den_r = a + b
    num_r = a * a + b * b
    den = jnp.dot(den_r, p, preferred_element_type=jnp.float32)
    num = jnp.dot(num_r, p, preferred_element_type=jnp.float32)
    return num / jnp.where(den == 0.0, 1.0, den)


def _pool_body(x1_ref, x2_ref, p_ref, o_ref):
    p = p_ref[...]
    r = x1_ref.shape[0]
    o_ref[:r, :] = _pool_half(jnp.maximum(x1_ref[...], 0.0), p)
    o_ref[r:, :] = _pool_half(jnp.maximum(x2_ref[...], 0.0), p)


def kernel(tensor):
    B, C, H, W = tensor.shape
    x = tensor.reshape(B * C * (H // 2), 2 * W)
    rows = x.shape[0]
    grid = rows // (2 * _ROWS)
    pairs = (jnp.arange(W)[:, None] // 2 == jnp.arange(W // 2)[None, :])
    p = pairs.astype(jnp.float32)
    out = pl.pallas_call(
        _pool_body,
        grid=(grid,),
        in_specs=[
            pl.BlockSpec((_ROWS, 2 * W), lambda i: (2 * i, 0)),
            pl.BlockSpec((_ROWS, 2 * W), lambda i: (2 * i + 1, 0)),
            pl.BlockSpec((W, W // 2), lambda i: (0, 0)),
        ],
        out_specs=pl.BlockSpec((2 * _ROWS, W // 2), lambda i: (i, 0)),
        out_shape=jax.ShapeDtypeStruct((rows, W // 2), jnp.float32),
    )(x, x, p)
    return out.reshape(B, C, H // 2, W // 2)
